# baseline (speedup 1.0000x reference)
import functools
import math

import jax
import jax.numpy as jnp
from jax.experimental import pallas as pl
from jax.experimental.pallas import tpu as pltpu

_D_P, _D_T, _D_F = 128, 256, 384
_D = 768
_NUM_PART, _NUM_TIMES, _NUM_F0 = 64, 601, 360
_NHEAD, _NLAYERS, _DFF = 12, 6, 2048
_CTX, _PRED = 1024, 64
_S = _CTX + _PRED
_B = 8
_HD = _D // _NHEAD
_NEG = -1e9

_f32 = jnp.float32


def _dot_t(a, b):
    return jax.lax.dot_general(a, b, (((1,), (1,)), ((), ())),
                               preferred_element_type=_f32)


def _dot_tl(a, b):
    return jax.lax.dot_general(a, b, (((0,), (0,)), ((), ())),
                               preferred_element_type=_f32)


def _layernorm(y, g, b):
    mu = jnp.mean(y, axis=-1, keepdims=True)
    c = y - mu
    var = jnp.mean(c * c, axis=-1, keepdims=True)
    return c * jax.lax.rsqrt(var + 1e-5) * g + b


def _embed_body(idxp_ref, time_ref, idxf_ref, part_ref, f0_ref, pad_ref,
                x_ref):
    idxp = idxp_ref[0]
    rows_p = jax.lax.broadcasted_iota(jnp.int32, (_NUM_PART, _CTX), 0)
    oh_p = jnp.where(rows_p == idxp, 1.0, 0.0).astype(_f32)
    x_ref[0, 0:_CTX, 0:_D_P] = _dot_tl(oh_p, part_ref[...])

    t = time_ref[0]
    ridx = jax.lax.broadcasted_iota(jnp.int32, (_D_T, _CTX), 0)
    i2 = (ridx >> 1).astype(_f32)
    wk = jnp.exp2(i2 * (-math.log2(10000.0) / _D_T)) * (_D_T / _NUM_TIMES)
    ang = wk * t
    embT = jnp.where((ridx & 1) == 0, jnp.sin(ang), jnp.cos(ang))
    eri = jax.lax.broadcasted_iota(jnp.int32, (_D_T, _D_T), 0)
    eci = jax.lax.broadcasted_iota(jnp.int32, (_D_T, _D_T), 1)
    eye = jnp.where(eri == eci, 1.0, 0.0).astype(_f32)
    x_ref[0, 0:_CTX, _D_P:_D_P + _D_T] = _dot_tl(embT, eye)

    idxf = idxf_ref[0]
    rows_f = jax.lax.broadcasted_iota(jnp.int32, (_NUM_F0, _CTX), 0)
    oh_f = jnp.where(rows_f == idxf, 1.0, 0.0).astype(_f32)
    x_ref[0, 0:_CTX, _D_P + _D_T:_D] = _dot_tl(oh_f, f0_ref[...])

    x_ref[0, _CTX:_S, :] = jnp.broadcast_to(pad_ref[...], (_PRED, _D))


def _layer_body(x_ref, wqkv_ref, bqkv_ref, wo_ref, bo_ref, g1_ref, be1_ref,
                w1_ref, bf1_ref, w2_ref, bf2_ref, g2_ref, be2_ref,
                out_ref, qkv_s, attn_s):
    scale = 1.0 / math.sqrt(float(_HD))

    def _qkv_chunk(i, _):
        r0 = i * 136
        xi = x_ref[0, pl.ds(r0, 136), :]
        for j in range(3):
            c0 = j * _D
            acc = _dot_t(xi, wqkv_ref[c0:c0 + _D, :]) + bqkv_ref[:, c0:c0 + _D]
            qkv_s[pl.ds(r0, 136), c0:c0 + _D] = acc
        return 0

    jax.lax.fori_loop(0, 8, _qkv_chunk, 0)

    for h in range(12):
        k_h = qkv_s[:, _D + h * _HD:_D + (h + 1) * _HD]
        v_h = qkv_s[:, 2 * _D + h * _HD:2 * _D + (h + 1) * _HD]

        def _attn_chunk(i, _, k_h=k_h, v_h=v_h, h=h):
            r0 = i * 272
            q = qkv_s[pl.ds(r0, 272), h * _HD:(h + 1) * _HD]
            s = _dot_t(q, k_h) * scale
            row = r0 + jax.lax.broadcasted_iota(jnp.int32, (272, _S), 0)
            col = jax.lax.broadcasted_iota(jnp.int32, (272, _S), 1)
            lim = jnp.minimum(row, _CTX - 1)
            s = jnp.where(col <= lim, s, _NEG)
            m = jnp.max(s, axis=-1, keepdims=True)
            e = jnp.exp(s - m)
            den = jnp.sum(e, axis=-1, keepdims=True)
            p = e * (1.0 / den)
            o = jax.lax.dot_general(p, v_h, (((1,), (0,)), ((), ())),
                                    preferred_element_type=_f32)
            attn_s[pl.ds(r0, 272), h * _HD:(h + 1) * _HD] = o
            return 0

        jax.lax.fori_loop(0, 4, _attn_chunk, 0)

    def _mlp_chunk(i, _):
        r0 = i * 136
        a = attn_s[pl.ds(r0, 136), :]
        proj = _dot_t(a, wo_ref[...]) + bo_ref[...]
        y = _layernorm(x_ref[0, pl.ds(r0, 136), :] + proj,
                       g1_ref[...], be1_ref[...])
        f = jnp.maximum(_dot_t(y, w1_ref[...]) + bf1_ref[...], 0.0)
        f2 = _dot_t(f, w2_ref[...]) + bf2_ref[...]
        out_ref[0, pl.ds(r0, 136), :] = _layernorm(y + f2,
                                                   g2_ref[...], be2_ref[...])
        return 0

    jax.lax.fori_loop(0, 8, _mlp_chunk, 0)


def _head_body(x_ref, part_ref, time_ref, f0_ref, cat_ref, tl_ref, fl_ref):
    p = x_ref[0]
    cat_ref[0] = _dot_t(p[:, 0:_D_P], part_ref[...])
    tl_ref[0] = _dot_t(p[:, _D_P:_D_P + _D_T], time_ref[...])
    fl_ref[0] = _dot_t(p[:, _D_P + _D_T:_D], f0_ref[...])


_PARAMS = pltpu.CompilerParams(dimension_semantics=(pltpu.PARALLEL,),
                               vmem_limit_bytes=100 * 1024 * 1024)


def _full(shape):
    return pl.BlockSpec(shape, lambda b: (0,) * len(shape))


def _embed(idxp, time, idxf, part_table, f0_table, pad):
    return pl.pallas_call(
        _embed_body,
        grid=(_B,),
        in_specs=[
            pl.BlockSpec((1, 1, _CTX), lambda b: (b, 0, 0)),
            pl.BlockSpec((1, 1, _CTX), lambda b: (b, 0, 0)),
            pl.BlockSpec((1, 1, _CTX), lambda b: (b, 0, 0)),
            _full((_NUM_PART, _D_P)),
            _full((_NUM_F0, _D_F)),
            _full((1, _D)),
        ],
        out_specs=pl.BlockSpec((1, _S, _D), lambda b: (b, 0, 0)),
        out_shape=jax.ShapeDtypeStruct((_B, _S, _D), _f32),
        compiler_params=_PARAMS,
    )(idxp, time, idxf, part_table, f0_table, pad)


def _layer(x, wqkv, bqkv, wo, bo, g1, be1, w1, bf1, w2, bf2, g2, be2):
    return pl.pallas_call(
        _layer_body,
        grid=(_B,),
        in_specs=[
            pl.BlockSpec((1, _S, _D), lambda b: (b, 0, 0)),
            _full((3 * _D, _D)),
            _full((1, 3 * _D)),
            _full((_D, _D)),
            _full((1, _D)),
            _full((1, _D)),
            _full((1, _D)),
            _full((_DFF, _D)),
            _full((1, _DFF)),
            _full((_D, _DFF)),
            _full((1, _D)),
            _full((1, _D)),
            _full((1, _D)),
        ],
        out_specs=pl.BlockSpec((1, _S, _D), lambda b: (b, 0, 0)),
        out_shape=jax.ShapeDtypeStruct((_B, _S, _D), _f32),
        scratch_shapes=[
            pltpu.VMEM((_S, 3 * _D), _f32),
            pltpu.VMEM((_S, _D), _f32),
        ],
        compiler_params=_PARAMS,
    )(x, wqkv, bqkv, wo, bo, g1, be1, w1, bf1, w2, bf2, g2, be2)


def _head(x, part_table, time_table, f0_table):
    return pl.pallas_call(
        _head_body,
        grid=(_B,),
        in_specs=[
            pl.BlockSpec((1, _PRED, _D), lambda b: (b, _CTX // _PRED, 0)),
            _full((_NUM_PART, _D_P)),
            _full((_NUM_TIMES, _D_T)),
            _full((_NUM_F0, _D_F)),
        ],
        out_specs=[
            pl.BlockSpec((1, _PRED, _NUM_PART), lambda b: (b, 0, 0)),
            pl.BlockSpec((1, _PRED, _NUM_TIMES), lambda b: (b, 0, 0)),
            pl.BlockSpec((1, _PRED, _NUM_F0), lambda b: (b, 0, 0)),
        ],
        out_shape=[
            jax.ShapeDtypeStruct((_B, _PRED, _NUM_PART), _f32),
            jax.ShapeDtypeStruct((_B, _PRED, _NUM_TIMES), _f32),
            jax.ShapeDtypeStruct((_B, _PRED, _NUM_F0), _f32),
        ],
        compiler_params=_PARAMS,
    )(x, part_table, time_table, f0_table)


def kernel(context_participant, context_time, context_f0, part_table,
           time_table, f0_table, pad, Wqkv, bqkv, Wo, bo, ln1_g, ln1_b,
           W1, b1, W2, b2, ln2_g, ln2_b):
    idxp = context_participant.astype(jnp.int32).reshape(_B, 1, _CTX)
    idxf = context_f0.astype(jnp.int32).reshape(_B, 1, _CTX)
    time = context_time.reshape(_B, 1, _CTX)

    x = _embed(idxp, time, idxf, part_table, f0_table, pad.reshape(1, _D))

    ws = (Wqkv, bqkv.reshape(_NLAYERS, 1, -1), Wo, bo.reshape(_NLAYERS, 1, -1),
          ln1_g.reshape(_NLAYERS, 1, -1), ln1_b.reshape(_NLAYERS, 1, -1),
          W1, b1.reshape(_NLAYERS, 1, -1), W2, b2.reshape(_NLAYERS, 1, -1),
          ln2_g.reshape(_NLAYERS, 1, -1), ln2_b.reshape(_NLAYERS, 1, -1))

    def _scan_body(xc, w):
        return _layer(xc, *w), None

    x, _ = jax.lax.scan(_scan_body, x, ws)

    return tuple(_head(x, part_table, time_table, f0_table))


# speedup vs baseline: 1.0218x; 1.0218x over previous
import functools
import math

import jax
import jax.numpy as jnp
from jax.experimental import pallas as pl
from jax.experimental.pallas import tpu as pltpu

_D_P, _D_T, _D_F = 128, 256, 384
_D = 768
_NUM_PART, _NUM_TIMES, _NUM_F0 = 64, 601, 360
_NHEAD, _NLAYERS, _DFF = 12, 6, 2048
_CTX, _PRED = 1024, 64
_S = _CTX + _PRED
_B = 8
_HD = _D // _NHEAD
_NEG = -1e9

_f32 = jnp.float32


def _dot_t(a, b):
    return jax.lax.dot_general(a, b, (((1,), (1,)), ((), ())),
                               preferred_element_type=_f32)


def _dot_tl(a, b):
    return jax.lax.dot_general(a, b, (((0,), (0,)), ((), ())),
                               preferred_element_type=_f32)


def _layernorm(y, g, b):
    mu = jnp.mean(y, axis=-1, keepdims=True)
    c = y - mu
    var = jnp.mean(c * c, axis=-1, keepdims=True)
    return c * jax.lax.rsqrt(var + 1e-5) * g + b


def _embed_body(idxp_ref, time_ref, idxf_ref, part_ref, f0_ref, pad_ref,
                x_ref):
    idxp = idxp_ref[0]
    rows_p = jax.lax.broadcasted_iota(jnp.int32, (_NUM_PART, _CTX), 0)
    oh_p = jnp.where(rows_p == idxp, 1.0, 0.0).astype(_f32)
    x_ref[0, 0:_CTX, 0:_D_P] = _dot_tl(oh_p, part_ref[...])

    t = time_ref[0]
    ridx = jax.lax.broadcasted_iota(jnp.int32, (_D_T, _CTX), 0)
    i2 = (ridx >> 1).astype(_f32)
    wk = jnp.exp2(i2 * (-math.log2(10000.0) / _D_T)) * (_D_T / _NUM_TIMES)
    ang = wk * t
    embT = jnp.where((ridx & 1) == 0, jnp.sin(ang), jnp.cos(ang))
    eri = jax.lax.broadcasted_iota(jnp.int32, (_D_T, _D_T), 0)
    eci = jax.lax.broadcasted_iota(jnp.int32, (_D_T, _D_T), 1)
    eye = jnp.where(eri == eci, 1.0, 0.0).astype(_f32)
    x_ref[0, 0:_CTX, _D_P:_D_P + _D_T] = _dot_tl(embT, eye)

    idxf = idxf_ref[0]
    rows_f = jax.lax.broadcasted_iota(jnp.int32, (_NUM_F0, _CTX), 0)
    oh_f = jnp.where(rows_f == idxf, 1.0, 0.0).astype(_f32)
    x_ref[0, 0:_CTX, _D_P + _D_T:_D] = _dot_tl(oh_f, f0_ref[...])

    x_ref[0, _CTX:_S, :] = jnp.broadcast_to(pad_ref[...], (_PRED, _D))


def _layer_body(x_ref, wqkv_ref, bqkv_ref, wo_ref, bo_ref, g1_ref, be1_ref,
                w1_ref, bf1_ref, w2_ref, bf2_ref, g2_ref, be2_ref,
                out_ref, qkv_s, attn_s):
    scale = 1.0 / math.sqrt(float(_HD))
    bf16 = jnp.bfloat16

    def _qkv_chunk(i, _):
        r0 = i * 272
        xi = x_ref[0, pl.ds(r0, 272), :].astype(bf16)
        for j in range(3):
            c0 = j * _D
            acc = _dot_t(xi, wqkv_ref[c0:c0 + _D, :]) + bqkv_ref[:, c0:c0 + _D]
            qkv_s[pl.ds(r0, 272), c0:c0 + _D] = acc.astype(bf16)
        return 0

    jax.lax.fori_loop(0, 4, _qkv_chunk, 0)

    for h in range(12):
        k_h = qkv_s[:, _D + h * _HD:_D + (h + 1) * _HD]
        v_h = qkv_s[:, 2 * _D + h * _HD:2 * _D + (h + 1) * _HD]

        def _attn_chunk(i, _, k_h=k_h, v_h=v_h, h=h):
            r0 = i * 272
            q = qkv_s[pl.ds(r0, 272), h * _HD:(h + 1) * _HD]
            s = _dot_t(q, k_h) * scale
            row = r0 + jax.lax.broadcasted_iota(jnp.int32, (272, _S), 0)
            col = jax.lax.broadcasted_iota(jnp.int32, (272, _S), 1)
            lim = jnp.minimum(row, _CTX - 1)
            s = jnp.where(col <= lim, s, _NEG)
            m = jnp.max(s, axis=-1, keepdims=True)
            e = jnp.exp(s - m)
            den = jnp.sum(e, axis=-1, keepdims=True)
            p = (e * (1.0 / den)).astype(jnp.bfloat16)
            o = jax.lax.dot_general(p, v_h, (((1,), (0,)), ((), ())),
                                    preferred_element_type=_f32)
            attn_s[pl.ds(r0, 272), h * _HD:(h + 1) * _HD] = o
            return 0

        jax.lax.fori_loop(0, 4, _attn_chunk, 0)

    def _mlp_chunk(i, _):
        r0 = i * 136
        a = attn_s[pl.ds(r0, 136), :].astype(bf16)
        proj = _dot_t(a, wo_ref[...]) + bo_ref[...]
        y = _layernorm(x_ref[0, pl.ds(r0, 136), :] + proj,
                       g1_ref[...], be1_ref[...])
        f = jnp.maximum(_dot_t(y.astype(bf16), w1_ref[...]) + bf1_ref[...],
                        0.0)
        f2 = _dot_t(f.astype(bf16), w2_ref[...]) + bf2_ref[...]
        out_ref[0, pl.ds(r0, 136), :] = _layernorm(y + f2,
                                                   g2_ref[...], be2_ref[...])
        return 0

    jax.lax.fori_loop(0, 8, _mlp_chunk, 0)


def _head_body(x_ref, part_ref, time_ref, f0_ref, cat_ref, tl_ref, fl_ref):
    p = x_ref[0]
    cat_ref[0] = _dot_t(p[:, 0:_D_P], part_ref[...])
    tl_ref[0] = _dot_t(p[:, _D_P:_D_P + _D_T], time_ref[...])
    fl_ref[0] = _dot_t(p[:, _D_P + _D_T:_D], f0_ref[...])


_PARAMS = pltpu.CompilerParams(dimension_semantics=(pltpu.PARALLEL,),
                               vmem_limit_bytes=100 * 1024 * 1024)


def _full(shape):
    return pl.BlockSpec(shape, lambda b: (0,) * len(shape))


def _embed(idxp, time, idxf, part_table, f0_table, pad):
    return pl.pallas_call(
        _embed_body,
        grid=(_B,),
        in_specs=[
            pl.BlockSpec((1, 1, _CTX), lambda b: (b, 0, 0)),
            pl.BlockSpec((1, 1, _CTX), lambda b: (b, 0, 0)),
            pl.BlockSpec((1, 1, _CTX), lambda b: (b, 0, 0)),
            _full((_NUM_PART, _D_P)),
            _full((_NUM_F0, _D_F)),
            _full((1, _D)),
        ],
        out_specs=pl.BlockSpec((1, _S, _D), lambda b: (b, 0, 0)),
        out_shape=jax.ShapeDtypeStruct((_B, _S, _D), _f32),
        compiler_params=_PARAMS,
    )(idxp, time, idxf, part_table, f0_table, pad)


def _layer(x, wqkv, bqkv, wo, bo, g1, be1, w1, bf1, w2, bf2, g2, be2):
    return pl.pallas_call(
        _layer_body,
        grid=(_B,),
        in_specs=[
            pl.BlockSpec((1, _S, _D), lambda b: (b, 0, 0)),
            _full((3 * _D, _D)),
            _full((1, 3 * _D)),
            _full((_D, _D)),
            _full((1, _D)),
            _full((1, _D)),
            _full((1, _D)),
            _full((_DFF, _D)),
            _full((1, _DFF)),
            _full((_D, _DFF)),
            _full((1, _D)),
            _full((1, _D)),
            _full((1, _D)),
        ],
        out_specs=pl.BlockSpec((1, _S, _D), lambda b: (b, 0, 0)),
        out_shape=jax.ShapeDtypeStruct((_B, _S, _D), _f32),
        scratch_shapes=[
            pltpu.VMEM((_S, 3 * _D), jnp.bfloat16),
            pltpu.VMEM((_S, _D), _f32),
        ],
        compiler_params=_PARAMS,
    )(x, wqkv, bqkv, wo, bo, g1, be1, w1, bf1, w2, bf2, g2, be2)


def _head(x, part_table, time_table, f0_table):
    return pl.pallas_call(
        _head_body,
        grid=(_B,),
        in_specs=[
            pl.BlockSpec((1, _PRED, _D), lambda b: (b, _CTX // _PRED, 0)),
            _full((_NUM_PART, _D_P)),
            _full((_NUM_TIMES, _D_T)),
            _full((_NUM_F0, _D_F)),
        ],
        out_specs=[
            pl.BlockSpec((1, _PRED, _NUM_PART), lambda b: (b, 0, 0)),
            pl.BlockSpec((1, _PRED, _NUM_TIMES), lambda b: (b, 0, 0)),
            pl.BlockSpec((1, _PRED, _NUM_F0), lambda b: (b, 0, 0)),
        ],
        out_shape=[
            jax.ShapeDtypeStruct((_B, _PRED, _NUM_PART), _f32),
            jax.ShapeDtypeStruct((_B, _PRED, _NUM_TIMES), _f32),
            jax.ShapeDtypeStruct((_B, _PRED, _NUM_F0), _f32),
        ],
        compiler_params=_PARAMS,
    )(x, part_table, time_table, f0_table)


def kernel(context_participant, context_time, context_f0, part_table,
           time_table, f0_table, pad, Wqkv, bqkv, Wo, bo, ln1_g, ln1_b,
           W1, b1, W2, b2, ln2_g, ln2_b):
    idxp = context_participant.astype(jnp.int32).reshape(_B, 1, _CTX)
    idxf = context_f0.astype(jnp.int32).reshape(_B, 1, _CTX)
    time = context_time.reshape(_B, 1, _CTX)

    x = _embed(idxp, time, idxf, part_table, f0_table, pad.reshape(1, _D))

    bf16 = jnp.bfloat16
    Wqkv, Wo, W1, W2 = (Wqkv.astype(bf16), Wo.astype(bf16),
                        W1.astype(bf16), W2.astype(bf16))
    ws = (Wqkv, bqkv.reshape(_NLAYERS, 1, -1), Wo, bo.reshape(_NLAYERS, 1, -1),
          ln1_g.reshape(_NLAYERS, 1, -1), ln1_b.reshape(_NLAYERS, 1, -1),
          W1, b1.reshape(_NLAYERS, 1, -1), W2, b2.reshape(_NLAYERS, 1, -1),
          ln2_g.reshape(_NLAYERS, 1, -1), ln2_b.reshape(_NLAYERS, 1, -1))

    def _scan_body(xc, w):
        return _layer(xc, *w), None

    x, _ = jax.lax.scan(_scan_body, x, ws)

    return tuple(_head(x, part_table, time_table, f0_table))


# speedup vs baseline: 1.2682x; 1.2411x over previous
import functools
import math

import jax
import jax.numpy as jnp
from jax.experimental import pallas as pl
from jax.experimental.pallas import tpu as pltpu

_D_P, _D_T, _D_F = 128, 256, 384
_D = 768
_NUM_PART, _NUM_TIMES, _NUM_F0 = 64, 601, 360
_NHEAD, _NLAYERS, _DFF = 12, 6, 2048
_CTX, _PRED = 1024, 64
_S = _CTX + _PRED
_B = 8
_HD = _D // _NHEAD
_NEG = -1e9

_f32 = jnp.float32


def _dot_t(a, b):
    return jax.lax.dot_general(a, b, (((1,), (1,)), ((), ())),
                               preferred_element_type=_f32)


def _dot_tl(a, b):
    return jax.lax.dot_general(a, b, (((0,), (0,)), ((), ())),
                               preferred_element_type=_f32)


def _layernorm(y, g, b):
    mu = jnp.mean(y, axis=-1, keepdims=True)
    c = y - mu
    var = jnp.mean(c * c, axis=-1, keepdims=True)
    return c * jax.lax.rsqrt(var + 1e-5) * g + b


def _embed_body(idxp_ref, time_ref, idxf_ref, part_ref, f0_ref, pad_ref,
                x_ref):
    idxp = idxp_ref[0]
    rows_p = jax.lax.broadcasted_iota(jnp.int32, (_NUM_PART, _CTX), 0)
    oh_p = jnp.where(rows_p == idxp, 1.0, 0.0).astype(_f32)
    x_ref[0, 0:_CTX, 0:_D_P] = _dot_tl(oh_p, part_ref[...])

    t = time_ref[0]
    ridx = jax.lax.broadcasted_iota(jnp.int32, (_D_T, _CTX), 0)
    i2 = (ridx >> 1).astype(_f32)
    wk = jnp.exp2(i2 * (-math.log2(10000.0) / _D_T)) * (_D_T / _NUM_TIMES)
    ang = wk * t
    embT = jnp.where((ridx & 1) == 0, jnp.sin(ang), jnp.cos(ang))
    eri = jax.lax.broadcasted_iota(jnp.int32, (_D_T, _D_T), 0)
    eci = jax.lax.broadcasted_iota(jnp.int32, (_D_T, _D_T), 1)
    eye = jnp.where(eri == eci, 1.0, 0.0).astype(_f32)
    x_ref[0, 0:_CTX, _D_P:_D_P + _D_T] = _dot_tl(embT, eye)

    idxf = idxf_ref[0]
    rows_f = jax.lax.broadcasted_iota(jnp.int32, (_NUM_F0, _CTX), 0)
    oh_f = jnp.where(rows_f == idxf, 1.0, 0.0).astype(_f32)
    x_ref[0, 0:_CTX, _D_P + _D_T:_D] = _dot_tl(oh_f, f0_ref[...])

    x_ref[0, _CTX:_S, :] = jnp.broadcast_to(pad_ref[...], (_PRED, _D))


def _layer_body(x_ref, wqkv_ref, bqkv_ref, wo_ref, bo_ref, g1_ref, be1_ref,
                w1_ref, bf1_ref, w2_ref, bf2_ref, g2_ref, be2_ref,
                out_ref, qkv_s, attn_s, mask_s):
    scale = 1.0 / math.sqrt(float(_HD))
    bf16 = jnp.bfloat16

    def _qkv_chunk(i, _):
        r0 = i * 272
        xi = x_ref[0, pl.ds(r0, 272), :].astype(bf16)
        for j in range(3):
            c0 = j * _D
            acc = _dot_t(xi, wqkv_ref[c0:c0 + _D, :]) + bqkv_ref[:, c0:c0 + _D]
            if j == 0:
                acc = acc * scale
            qkv_s[pl.ds(r0, 272), c0:c0 + _D] = acc.astype(bf16)
        row = r0 + jax.lax.broadcasted_iota(jnp.int32, (272, _S), 0)
        col = jax.lax.broadcasted_iota(jnp.int32, (272, _S), 1)
        lim = jnp.minimum(row, _CTX - 1)
        mask_s[pl.ds(r0, 272), :] = jnp.where(col <= lim, 0.0, _NEG)
        return 0

    jax.lax.fori_loop(0, 4, _qkv_chunk, 0)

    for hp in range(6):
        def _attn_chunk(i, _, hp=hp):
            r0 = i * 272
            madd = mask_s[pl.ds(r0, 272), :]
            for h in (2 * hp, 2 * hp + 1):
                k_h = qkv_s[:, _D + h * _HD:_D + (h + 1) * _HD]
                v_h = qkv_s[:, 2 * _D + h * _HD:2 * _D + (h + 1) * _HD]
                q = qkv_s[pl.ds(r0, 272), h * _HD:(h + 1) * _HD]
                s = _dot_t(q, k_h) + madd
                m = jnp.max(s, axis=-1, keepdims=True)
                e = jnp.exp(s - m)
                den = jnp.sum(e, axis=-1, keepdims=True)
                o = jax.lax.dot_general(e.astype(bf16), v_h,
                                        (((1,), (0,)), ((), ())),
                                        preferred_element_type=_f32)
                o = o * (1.0 / den)
                attn_s[pl.ds(r0, 272), h * _HD:(h + 1) * _HD] = o
            return 0

        jax.lax.fori_loop(0, 4, _attn_chunk, 0)

    def _mlp_chunk(i, _):
        r0 = i * 136
        a = attn_s[pl.ds(r0, 136), :].astype(bf16)
        proj = _dot_t(a, wo_ref[...]) + bo_ref[...]
        y = _layernorm(x_ref[0, pl.ds(r0, 136), :] + proj,
                       g1_ref[...], be1_ref[...])
        f = jnp.maximum(_dot_t(y.astype(bf16), w1_ref[...]) + bf1_ref[...],
                        0.0)
        f2 = _dot_t(f.astype(bf16), w2_ref[...]) + bf2_ref[...]
        out_ref[0, pl.ds(r0, 136), :] = _layernorm(y + f2,
                                                   g2_ref[...], be2_ref[...])
        return 0

    jax.lax.fori_loop(0, 8, _mlp_chunk, 0)


def _head_body(x_ref, part_ref, time_ref, f0_ref, cat_ref, tl_ref, fl_ref):
    p = x_ref[0]
    cat_ref[0] = _dot_t(p[:, 0:_D_P], part_ref[...])
    tl_ref[0] = _dot_t(p[:, _D_P:_D_P + _D_T], time_ref[...])
    fl_ref[0] = _dot_t(p[:, _D_P + _D_T:_D], f0_ref[...])


_PARAMS = pltpu.CompilerParams(dimension_semantics=(pltpu.PARALLEL,),
                               vmem_limit_bytes=100 * 1024 * 1024)


def _full(shape):
    return pl.BlockSpec(shape, lambda b: (0,) * len(shape))


def _embed(idxp, time, idxf, part_table, f0_table, pad):
    return pl.pallas_call(
        _embed_body,
        grid=(_B,),
        in_specs=[
            pl.BlockSpec((1, 1, _CTX), lambda b: (b, 0, 0)),
            pl.BlockSpec((1, 1, _CTX), lambda b: (b, 0, 0)),
            pl.BlockSpec((1, 1, _CTX), lambda b: (b, 0, 0)),
            _full((_NUM_PART, _D_P)),
            _full((_NUM_F0, _D_F)),
            _full((1, _D)),
        ],
        out_specs=pl.BlockSpec((1, _S, _D), lambda b: (b, 0, 0)),
        out_shape=jax.ShapeDtypeStruct((_B, _S, _D), _f32),
        compiler_params=_PARAMS,
    )(idxp, time, idxf, part_table, f0_table, pad)


def _layer(x, wqkv, bqkv, wo, bo, g1, be1, w1, bf1, w2, bf2, g2, be2):
    return pl.pallas_call(
        _layer_body,
        grid=(_B,),
        in_specs=[
            pl.BlockSpec((1, _S, _D), lambda b: (b, 0, 0)),
            _full((3 * _D, _D)),
            _full((1, 3 * _D)),
            _full((_D, _D)),
            _full((1, _D)),
            _full((1, _D)),
            _full((1, _D)),
            _full((_DFF, _D)),
            _full((1, _DFF)),
            _full((_D, _DFF)),
            _full((1, _D)),
            _full((1, _D)),
            _full((1, _D)),
        ],
        out_specs=pl.BlockSpec((1, _S, _D), lambda b: (b, 0, 0)),
        out_shape=jax.ShapeDtypeStruct((_B, _S, _D), _f32),
        scratch_shapes=[
            pltpu.VMEM((_S, 3 * _D), jnp.bfloat16),
            pltpu.VMEM((_S, _D), _f32),
            pltpu.VMEM((_S, _S), _f32),
        ],
        compiler_params=_PARAMS,
    )(x, wqkv, bqkv, wo, bo, g1, be1, w1, bf1, w2, bf2, g2, be2)


def _head(x, part_table, time_table, f0_table):
    return pl.pallas_call(
        _head_body,
        grid=(_B,),
        in_specs=[
            pl.BlockSpec((1, _PRED, _D), lambda b: (b, _CTX // _PRED, 0)),
            _full((_NUM_PART, _D_P)),
            _full((_NUM_TIMES, _D_T)),
            _full((_NUM_F0, _D_F)),
        ],
        out_specs=[
            pl.BlockSpec((1, _PRED, _NUM_PART), lambda b: (b, 0, 0)),
            pl.BlockSpec((1, _PRED, _NUM_TIMES), lambda b: (b, 0, 0)),
            pl.BlockSpec((1, _PRED, _NUM_F0), lambda b: (b, 0, 0)),
        ],
        out_shape=[
            jax.ShapeDtypeStruct((_B, _PRED, _NUM_PART), _f32),
            jax.ShapeDtypeStruct((_B, _PRED, _NUM_TIMES), _f32),
            jax.ShapeDtypeStruct((_B, _PRED, _NUM_F0), _f32),
        ],
        compiler_params=_PARAMS,
    )(x, part_table, time_table, f0_table)


def kernel(context_participant, context_time, context_f0, part_table,
           time_table, f0_table, pad, Wqkv, bqkv, Wo, bo, ln1_g, ln1_b,
           W1, b1, W2, b2, ln2_g, ln2_b):
    idxp = context_participant.astype(jnp.int32).reshape(_B, 1, _CTX)
    idxf = context_f0.astype(jnp.int32).reshape(_B, 1, _CTX)
    time = context_time.reshape(_B, 1, _CTX)

    x = _embed(idxp, time, idxf, part_table, f0_table, pad.reshape(1, _D))

    bf16 = jnp.bfloat16
    Wqkv, Wo, W1, W2 = (Wqkv.astype(bf16), Wo.astype(bf16),
                        W1.astype(bf16), W2.astype(bf16))
    ws = (Wqkv, bqkv.reshape(_NLAYERS, 1, -1), Wo, bo.reshape(_NLAYERS, 1, -1),
          ln1_g.reshape(_NLAYERS, 1, -1), ln1_b.reshape(_NLAYERS, 1, -1),
          W1, b1.reshape(_NLAYERS, 1, -1), W2, b2.reshape(_NLAYERS, 1, -1),
          ln2_g.reshape(_NLAYERS, 1, -1), ln2_b.reshape(_NLAYERS, 1, -1))

    def _scan_body(xc, w):
        return _layer(xc, *w), None

    x, _ = jax.lax.scan(_scan_body, x, ws)

    return tuple(_head(x, part_table, time_table, f0_table))


# speedup vs baseline: 1.3894x; 1.0955x over previous
import functools
import math

import jax
import jax.numpy as jnp
from jax.experimental import pallas as pl
from jax.experimental.pallas import tpu as pltpu

_D_P, _D_T, _D_F = 128, 256, 384
_D = 768
_NUM_PART, _NUM_TIMES, _NUM_F0 = 64, 601, 360
_NHEAD, _NLAYERS, _DFF = 12, 6, 2048
_CTX, _PRED = 1024, 64
_S = _CTX + _PRED
_B = 8
_HD = _D // _NHEAD
_NEG = -1e9

_f32 = jnp.float32


def _dot_t(a, b):
    return jax.lax.dot_general(a, b, (((1,), (1,)), ((), ())),
                               preferred_element_type=_f32)


def _dot_tl(a, b):
    return jax.lax.dot_general(a, b, (((0,), (0,)), ((), ())),
                               preferred_element_type=_f32)


def _layernorm(y, g, b):
    mu = jnp.mean(y, axis=-1, keepdims=True)
    c = y - mu
    var = jnp.mean(c * c, axis=-1, keepdims=True)
    return c * jax.lax.rsqrt(var + 1e-5) * g + b


def _embed_body(idxp_ref, time_ref, idxf_ref, part_ref, f0_ref, pad_ref,
                x_ref):
    idxp = idxp_ref[0]
    rows_p = jax.lax.broadcasted_iota(jnp.int32, (_NUM_PART, _CTX), 0)
    oh_p = jnp.where(rows_p == idxp, 1.0, 0.0).astype(_f32)
    x_ref[0, 0:_CTX, 0:_D_P] = _dot_tl(oh_p, part_ref[...])

    t = time_ref[0]
    ridx = jax.lax.broadcasted_iota(jnp.int32, (_D_T, _CTX), 0)
    i2 = (ridx >> 1).astype(_f32)
    wk = jnp.exp2(i2 * (-math.log2(10000.0) / _D_T)) * (_D_T / _NUM_TIMES)
    ang = wk * t
    embT = jnp.where((ridx & 1) == 0, jnp.sin(ang), jnp.cos(ang))
    eri = jax.lax.broadcasted_iota(jnp.int32, (_D_T, _D_T), 0)
    eci = jax.lax.broadcasted_iota(jnp.int32, (_D_T, _D_T), 1)
    eye = jnp.where(eri == eci, 1.0, 0.0).astype(_f32)
    x_ref[0, 0:_CTX, _D_P:_D_P + _D_T] = _dot_tl(embT, eye)

    idxf = idxf_ref[0]
    rows_f = jax.lax.broadcasted_iota(jnp.int32, (_NUM_F0, _CTX), 0)
    oh_f = jnp.where(rows_f == idxf, 1.0, 0.0).astype(_f32)
    x_ref[0, 0:_CTX, _D_P + _D_T:_D] = _dot_tl(oh_f, f0_ref[...])

    x_ref[0, _CTX:_S, :] = jnp.broadcast_to(pad_ref[...], (_PRED, _D))


def _layer_body(x_ref, wqkv_ref, bqkv_ref, wo_ref, bo_ref, g1_ref, be1_ref,
                w1_ref, bf1_ref, w2_ref, bf2_ref, g2_ref, be2_ref,
                out_ref, qkv_s, attn_s, mask_s):
    scale = 1.0 / math.sqrt(float(_HD))
    bf16 = jnp.bfloat16

    def _qkv_chunk(i, _):
        r0 = i * 272
        xi = x_ref[0, pl.ds(r0, 272), :].astype(bf16)
        for j in range(3):
            c0 = j * _D
            acc = _dot_t(xi, wqkv_ref[c0:c0 + _D, :]) + bqkv_ref[:, c0:c0 + _D]
            if j == 0:
                acc = acc * scale
            qkv_s[pl.ds(r0, 272), c0:c0 + _D] = acc.astype(bf16)
        row = r0 + jax.lax.broadcasted_iota(jnp.int32, (272, _S), 0)
        col = jax.lax.broadcasted_iota(jnp.int32, (272, _S), 1)
        lim = jnp.minimum(row, _CTX - 1)
        mask_s[pl.ds(r0, 272), :] = jnp.where(col <= lim, 0.0, _NEG)
        return 0

    jax.lax.fori_loop(0, 4, _qkv_chunk, 0)

    def _mk_attn(hp, base, w):
        def _attn_chunk(i, _):
            r0 = base + i * 272
            madd = mask_s[pl.ds(r0, 272), 0:w]
            for h in (2 * hp, 2 * hp + 1):
                k_h = qkv_s[0:w, _D + h * _HD:_D + (h + 1) * _HD]
                v_h = qkv_s[0:w, 2 * _D + h * _HD:2 * _D + (h + 1) * _HD]
                q = qkv_s[pl.ds(r0, 272), h * _HD:(h + 1) * _HD]
                s = _dot_t(q, k_h) + madd
                m = jnp.max(s, axis=-1, keepdims=True)
                e = jnp.exp(s - m)
                den = jnp.sum(e, axis=-1, keepdims=True)
                o = jax.lax.dot_general(e.astype(bf16), v_h,
                                        (((1,), (0,)), ((), ())),
                                        preferred_element_type=_f32)
                o = o * (1.0 / den)
                attn_s[pl.ds(r0, 272), h * _HD:(h + 1) * _HD] = o
            return 0

        return _attn_chunk

    for hp in range(6):
        jax.lax.fori_loop(0, 2, _mk_attn(hp, 0, 640), 0)
        jax.lax.fori_loop(0, 2, _mk_attn(hp, 544, 1088), 0)

    def _mlp_chunk(i, _):
        r0 = i * 136
        a = attn_s[pl.ds(r0, 136), :].astype(bf16)
        proj = _dot_t(a, wo_ref[...]) + bo_ref[...]
        y = _layernorm(x_ref[0, pl.ds(r0, 136), :] + proj,
                       g1_ref[...], be1_ref[...])
        f = jnp.maximum(_dot_t(y.astype(bf16), w1_ref[...]) + bf1_ref[...],
                        0.0)
        f2 = _dot_t(f.astype(bf16), w2_ref[...]) + bf2_ref[...]
        out_ref[0, pl.ds(r0, 136), :] = _layernorm(y + f2,
                                                   g2_ref[...], be2_ref[...])
        return 0

    jax.lax.fori_loop(0, 8, _mlp_chunk, 0)


def _head_body(x_ref, part_ref, time_ref, f0_ref, cat_ref, tl_ref, fl_ref):
    p = x_ref[0]
    cat_ref[0] = _dot_t(p[:, 0:_D_P], part_ref[...])
    tl_ref[0] = _dot_t(p[:, _D_P:_D_P + _D_T], time_ref[...])
    fl_ref[0] = _dot_t(p[:, _D_P + _D_T:_D], f0_ref[...])


_PARAMS = pltpu.CompilerParams(dimension_semantics=(pltpu.PARALLEL,),
                               vmem_limit_bytes=100 * 1024 * 1024)


def _full(shape):
    return pl.BlockSpec(shape, lambda b: (0,) * len(shape))


def _embed(idxp, time, idxf, part_table, f0_table, pad):
    return pl.pallas_call(
        _embed_body,
        grid=(_B,),
        in_specs=[
            pl.BlockSpec((1, 1, _CTX), lambda b: (b, 0, 0)),
            pl.BlockSpec((1, 1, _CTX), lambda b: (b, 0, 0)),
            pl.BlockSpec((1, 1, _CTX), lambda b: (b, 0, 0)),
            _full((_NUM_PART, _D_P)),
            _full((_NUM_F0, _D_F)),
            _full((1, _D)),
        ],
        out_specs=pl.BlockSpec((1, _S, _D), lambda b: (b, 0, 0)),
        out_shape=jax.ShapeDtypeStruct((_B, _S, _D), _f32),
        compiler_params=_PARAMS,
    )(idxp, time, idxf, part_table, f0_table, pad)


def _layer(x, wqkv, bqkv, wo, bo, g1, be1, w1, bf1, w2, bf2, g2, be2):
    return pl.pallas_call(
        _layer_body,
        grid=(_B,),
        in_specs=[
            pl.BlockSpec((1, _S, _D), lambda b: (b, 0, 0)),
            _full((3 * _D, _D)),
            _full((1, 3 * _D)),
            _full((_D, _D)),
            _full((1, _D)),
            _full((1, _D)),
            _full((1, _D)),
            _full((_DFF, _D)),
            _full((1, _DFF)),
            _full((_D, _DFF)),
            _full((1, _D)),
            _full((1, _D)),
            _full((1, _D)),
        ],
        out_specs=pl.BlockSpec((1, _S, _D), lambda b: (b, 0, 0)),
        out_shape=jax.ShapeDtypeStruct((_B, _S, _D), _f32),
        scratch_shapes=[
            pltpu.VMEM((_S, 3 * _D), jnp.bfloat16),
            pltpu.VMEM((_S, _D), _f32),
            pltpu.VMEM((_S, _S), _f32),
        ],
        compiler_params=_PARAMS,
    )(x, wqkv, bqkv, wo, bo, g1, be1, w1, bf1, w2, bf2, g2, be2)


def _head(x, part_table, time_table, f0_table):
    return pl.pallas_call(
        _head_body,
        grid=(_B,),
        in_specs=[
            pl.BlockSpec((1, _PRED, _D), lambda b: (b, _CTX // _PRED, 0)),
            _full((_NUM_PART, _D_P)),
            _full((_NUM_TIMES, _D_T)),
            _full((_NUM_F0, _D_F)),
        ],
        out_specs=[
            pl.BlockSpec((1, _PRED, _NUM_PART), lambda b: (b, 0, 0)),
            pl.BlockSpec((1, _PRED, _NUM_TIMES), lambda b: (b, 0, 0)),
            pl.BlockSpec((1, _PRED, _NUM_F0), lambda b: (b, 0, 0)),
        ],
        out_shape=[
            jax.ShapeDtypeStruct((_B, _PRED, _NUM_PART), _f32),
            jax.ShapeDtypeStruct((_B, _PRED, _NUM_TIMES), _f32),
            jax.ShapeDtypeStruct((_B, _PRED, _NUM_F0), _f32),
        ],
        compiler_params=_PARAMS,
    )(x, part_table, time_table, f0_table)


def kernel(context_participant, context_time, context_f0, part_table,
           time_table, f0_table, pad, Wqkv, bqkv, Wo, bo, ln1_g, ln1_b,
           W1, b1, W2, b2, ln2_g, ln2_b):
    idxp = context_participant.astype(jnp.int32).reshape(_B, 1, _CTX)
    idxf = context_f0.astype(jnp.int32).reshape(_B, 1, _CTX)
    time = context_time.reshape(_B, 1, _CTX)

    x = _embed(idxp, time, idxf, part_table, f0_table, pad.reshape(1, _D))

    bf16 = jnp.bfloat16
    Wqkv, Wo, W1, W2 = (Wqkv.astype(bf16), Wo.astype(bf16),
                        W1.astype(bf16), W2.astype(bf16))
    ws = (Wqkv, bqkv.reshape(_NLAYERS, 1, -1), Wo, bo.reshape(_NLAYERS, 1, -1),
          ln1_g.reshape(_NLAYERS, 1, -1), ln1_b.reshape(_NLAYERS, 1, -1),
          W1, b1.reshape(_NLAYERS, 1, -1), W2, b2.reshape(_NLAYERS, 1, -1),
          ln2_g.reshape(_NLAYERS, 1, -1), ln2_b.reshape(_NLAYERS, 1, -1))

    def _scan_body(xc, w):
        return _layer(xc, *w), None

    x, _ = jax.lax.scan(_scan_body, x, ws)

    return tuple(_head(x, part_table, time_table, f0_table))
